# baseline (speedup 1.0000x reference)
import jax
import jax.numpy as jnp
from jax import lax
from jax.experimental import pallas as pl
from jax.experimental.pallas import tpu as pltpu

_B, _H, _W = 8, 2048, 2048
_N = 64
_WIN = 128


def _probe_gather(b_s, y_s, xc_s, xc_v, x_hbm, out_ref, buf, sem):
    copies = []
    for i in range(_N):
        c0 = pl.multiple_of(xc_s[i] & ~(_WIN - 1), _WIN)
        copies.append(pltpu.make_async_copy(
            x_hbm.at[b_s[i], y_s[i], pl.ds(c0, _WIN)],
            buf.at[i], sem))
    for c in copies:
        c.start()
    lane = (xc_v[...] & (_WIN - 1)).reshape(_N, 1)
    cols = lax.broadcasted_iota(jnp.int32, (_N, _WIN), 1)
    mask = cols == lane
    pltpu.make_async_copy(
        x_hbm.at[0, pl.ds(0, _N), pl.ds(0, _WIN)], buf, sem).wait()
    picked = jnp.where(mask, buf[...], 0.0)
    out_ref[...] = jnp.sum(picked, axis=1)


def kernel(x, bidx, y, xc):
    return pl.pallas_call(
        _probe_gather,
        out_shape=jax.ShapeDtypeStruct((_N,), jnp.float32),
        grid_spec=pltpu.PrefetchScalarGridSpec(
            num_scalar_prefetch=3,
            in_specs=[
                pl.BlockSpec(memory_space=pltpu.VMEM),
                pl.BlockSpec(memory_space=pl.ANY),
            ],
            out_specs=pl.BlockSpec(memory_space=pltpu.VMEM),
            scratch_shapes=[
                pltpu.VMEM((_N, _WIN), jnp.float32),
                pltpu.SemaphoreType.DMA,
            ],
        ),
    )(bidx, y, xc, xc, x)


# speedup vs baseline: 1.2448x; 1.2448x over previous
import jax
import jax.numpy as jnp
from jax import lax
from jax.experimental import pallas as pl
from jax.experimental.pallas import tpu as pltpu

_B, _H, _W = 8, 2048, 2048
_N = 64
_WIN = 128


def _probe_gather(b_s, y_s, xc_s, xc_v, x_hbm, out_ref, buf, sem):
    copies = []
    for i in range(_N):
        c0 = pl.multiple_of(xc_s[i] & ~(_WIN - 1), _WIN)
        copies.append(pltpu.make_async_copy(
            x_hbm.at[b_s[i], y_s[i], pl.ds(c0, _WIN)],
            buf.at[i], sem))
    for c in copies:
        c.start()
    lane = (xc_v[...] & (_WIN - 1)).reshape(_N, 1)
    cols = lax.broadcasted_iota(jnp.int32, (_N, _WIN), 1)
    mask = cols == lane
    pltpu.make_async_copy(
        x_hbm.at[0, pl.ds(0, _N), pl.ds(0, _WIN)], buf, sem).wait()
    picked = jnp.where(mask, buf[...], 0.0)
    out_ref[...] = jnp.sum(picked, axis=1)


def kernel(x, bidx, y, xc):
    return pl.pallas_call(
        _probe_gather,
        out_shape=jax.ShapeDtypeStruct((_N,), jnp.float32),
        in_specs=[
            pl.BlockSpec(memory_space=pltpu.SMEM),
            pl.BlockSpec(memory_space=pltpu.SMEM),
            pl.BlockSpec(memory_space=pltpu.SMEM),
            pl.BlockSpec(memory_space=pltpu.VMEM),
            pl.BlockSpec(memory_space=pl.ANY),
        ],
        out_specs=pl.BlockSpec(memory_space=pltpu.VMEM),
        scratch_shapes=[
            pltpu.VMEM((_N, _WIN), jnp.float32),
            pltpu.SemaphoreType.DMA,
        ],
    )(bidx, y, xc, xc, x)
